# baseline (speedup 1.0000x reference)
import jax
import jax.numpy as jnp
from jax import lax
from jax.experimental import pallas as pl
from jax.experimental.pallas import tpu as pltpu
from jax.experimental.pallas import tpu_sc as plsc

N_NODES = 10000
N_EDGES = 320000
IN_FEATS = 128
HIDDEN = 16
OUT_FEATS = 2

NC, NS = 2, 16
NW = NC * NS
NP = 10240
NQ = NP // 8
XQ = N_NODES // 8
RPT = NP // NS
K = 128
EP = NW * 80 * K
EPW = EP // NW
NCHUNK = EPW // K
SBC = 16
NSB = NCHUNK // SBC


def _sc_mesh():
    return plsc.VectorSubcoreMesh(
        core_axis_name="c", subcore_axis_name="s",
        num_cores=NC, num_subcores=NS)


def _zero_acc_slice(zstage, acc_sh, s):
    z = jnp.zeros((HIDDEN,), jnp.float32)

    def zrow(i, carry):
        zstage[i, :] = z
        return carry
    lax.fori_loop(0, RPT, zrow, 0)
    pltpu.sync_copy(zstage, acc_sh.at[pl.ds(s * RPT, RPT)])


def _deg_body(e_hbm, ones_hbm, out_hbm, dst_v, ones_v, zstage, acc_sh, ssem):
    c = lax.axis_index("c")
    s = lax.axis_index("s")
    wid = s * NC + c
    _zero_acc_slice(zstage, acc_sh, s)
    pltpu.sync_copy(e_hbm.at[1].at[wid], dst_v)
    pltpu.sync_copy(ones_hbm, ones_v)
    plsc.subcore_barrier()

    def fire(j, carry):
        pltpu.async_copy(ones_v, acc_sh.at[dst_v.at[j]], ssem, add=True)
        return carry
    lax.fori_loop(0, NCHUNK, fire, 0)

    def drain(j, carry):
        pltpu.make_async_copy(ones_hbm, ones_v, ssem).wait()
        return carry
    lax.fori_loop(0, NCHUNK, drain, 0)

    plsc.subcore_barrier()
    pltpu.sync_copy(acc_sh.at[pl.ds(s * RPT, RPT)],
                    out_hbm.at[c].at[pl.ds(s * RPT, RPT)])


def _agg_body(rows_hbm, e_hbm, out_hbm,
              src_v, dst_v, bufs, zstage, acc_sh, gsems, ssems):
    c = lax.axis_index("c")
    s = lax.axis_index("s")
    wid = s * NC + c
    _zero_acc_slice(zstage, acc_sh, s)
    pltpu.sync_copy(e_hbm.at[0].at[wid], src_v)
    pltpu.sync_copy(e_hbm.at[1].at[wid], dst_v)
    plsc.subcore_barrier()

    def fire_gathers(sb, bi):
        base = sb * SBC
        def f(j, carry):
            pltpu.async_copy(rows_hbm.at[src_v.at[base + j]],
                             bufs.at[bi].at[j], gsems.at[bi])
            return carry
        lax.fori_loop(0, SBC, f, 0)

    def fire_scatters(sb, bi):
        base = sb * SBC
        def f(j, carry):
            pltpu.async_copy(bufs.at[bi].at[j], acc_sh.at[dst_v.at[base + j]],
                             ssems.at[bi], add=True)
            return carry
        lax.fori_loop(0, SBC, f, 0)

    def drain(sem):
        def f(j, carry):
            pltpu.make_async_copy(rows_hbm.at[pl.ds(0, K)],
                                  bufs.at[0].at[0], sem).wait()
            return carry
        lax.fori_loop(0, SBC, f, 0)

    fire_gathers(0, 0)
    for sb in range(NSB):
        bi = sb % 2
        drain(gsems.at[bi])
        if sb >= 1:
            drain(ssems.at[(sb - 1) % 2])
        if sb + 1 < NSB:
            fire_gathers(sb + 1, (sb + 1) % 2)
        fire_scatters(sb, bi)
    drain(ssems.at[(NSB - 1) % 2])

    plsc.subcore_barrier()
    pltpu.sync_copy(acc_sh.at[pl.ds(s * RPT, RPT)],
                    out_hbm.at[c].at[pl.ds(s * RPT, RPT)])


_sc_params = pltpu.CompilerParams(use_tc_tiling_on_sc=False)

_deg_call = pl.kernel(
    _deg_body,
    out_type=jax.ShapeDtypeStruct((NC, NP, HIDDEN), jnp.float32),
    mesh=_sc_mesh(),
    compiler_params=_sc_params,
    scratch_types=[
        pltpu.VMEM((NCHUNK, K), jnp.int32),
        pltpu.VMEM((K, HIDDEN), jnp.float32),
        pltpu.VMEM((RPT, HIDDEN), jnp.float32),
        pltpu.VMEM_SHARED((NP, HIDDEN), jnp.float32),
        pltpu.SemaphoreType.DMA,
    ],
)

_agg_call = pl.kernel(
    _agg_body,
    out_type=jax.ShapeDtypeStruct((NC, NP, HIDDEN), jnp.float32),
    mesh=_sc_mesh(),
    compiler_params=_sc_params,
    scratch_types=[
        pltpu.VMEM((NCHUNK, K), jnp.int32),
        pltpu.VMEM((NCHUNK, K), jnp.int32),
        pltpu.VMEM((2, SBC, K, HIDDEN), jnp.float32),
        pltpu.VMEM((RPT, HIDDEN), jnp.float32),
        pltpu.VMEM_SHARED((NP, HIDDEN), jnp.float32),
        pltpu.SemaphoreType.DMA((2,)),
        pltpu.SemaphoreType.DMA((2,)),
    ],
)


def _tc1_body(xq, w1b, degq, hsq, dinvq):
    dinv = lax.rsqrt(degq[0] + degq[1] + 1.0)
    dinvq[...] = dinv
    h = jnp.dot(xq[...], w1b[...], preferred_element_type=jnp.float32)
    hsq[pl.ds(0, XQ), :] = h * dinv[0:XQ, :]


def _tc2_body(a1p, hsq, dinvq, b1t, gsq):
    a1 = (a1p[0] + a1p[1] + hsq[...]) * dinvq[...] + b1t[...]
    gsq[...] = jnp.maximum(a1, 0.0) * dinvq[...]


def _tc3_body(a2p, gsq, dinvq, w2b, b2t, out):
    a2 = (a2p[0] + a2p[1] + gsq[...]) * dinvq[...]
    o = jnp.dot(a2, w2b[...], preferred_element_type=jnp.float32) + b2t[...]
    out[...] = o[0:XQ, :]


_tc1 = pl.pallas_call(
    _tc1_body,
    out_shape=[jax.ShapeDtypeStruct((NQ, 128), jnp.float32),
               jax.ShapeDtypeStruct((NQ, 128), jnp.float32)],
)

_tc2 = pl.pallas_call(
    _tc2_body,
    out_shape=jax.ShapeDtypeStruct((NQ, 128), jnp.float32),
)

_tc3 = pl.pallas_call(
    _tc3_body,
    out_shape=jax.ShapeDtypeStruct((XQ, 8 * OUT_FEATS), jnp.float32),
)


def kernel(x, edge_index, W1, b1, W2, b2):
    e = jnp.pad(edge_index.astype(jnp.int32), ((0, 0), (0, EP - N_EDGES)),
                constant_values=NP - 1)
    e = e.reshape(2, NW, NCHUNK, K)
    xq = x.reshape(XQ, 8 * IN_FEATS)
    w1b = jnp.kron(jnp.eye(8, dtype=jnp.float32), W1)
    w2b = jnp.kron(jnp.eye(8, dtype=jnp.float32), W2)
    b1t = jnp.tile(b1, 8).reshape(1, 128)
    b2t = jnp.tile(b2, 8).reshape(1, 8 * OUT_FEATS)
    onesKH = jnp.ones((K, HIDDEN), jnp.float32)

    degp = _deg_call(e, onesKH)
    hsq, dinvq = _tc1(xq, w1b, degp.reshape(NC, NQ, 128))
    a1p = _agg_call(hsq.reshape(NP, HIDDEN), e)
    gsq = _tc2(a1p.reshape(NC, NQ, 128), hsq, dinvq, b1t)
    a2p = _agg_call(gsq.reshape(NP, HIDDEN), e)
    outq = _tc3(a2p.reshape(NC, NQ, 128), gsq, dinvq, w2b, b2t)
    return outq.reshape(N_NODES, OUT_FEATS)


# speedup vs baseline: 1.8153x; 1.8153x over previous
import jax
import jax.numpy as jnp
from jax import lax
from jax.experimental import pallas as pl
from jax.experimental.pallas import tpu as pltpu
from jax.experimental.pallas import tpu_sc as plsc

N_NODES = 10000
N_EDGES = 320000
IN_FEATS = 128
HIDDEN = 16
OUT_FEATS = 2

NC, NS = 2, 16
NW = NC * NS
NP = 10240
NQ = NP // 8
XQ = N_NODES // 8
RPT = NP // NS
K = 128
EP = NW * 80 * K
EPW = EP // NW
NCHUNK = EPW // K
SBC = 16
NSB = NCHUNK // SBC


def _sc_mesh():
    return plsc.VectorSubcoreMesh(
        core_axis_name="c", subcore_axis_name="s",
        num_cores=NC, num_subcores=NS)


def _deg_body(e_hbm, ones_hbm, zeros_hbm, out_hbm, dst_v, ones_v, acc_sh,
              ssem):
    c = lax.axis_index("c")
    s = lax.axis_index("s")
    wid = s * NC + c
    pltpu.sync_copy(zeros_hbm, acc_sh.at[pl.ds(s * RPT, RPT)])
    pltpu.sync_copy(e_hbm.at[1].at[wid], dst_v)
    pltpu.sync_copy(ones_hbm, ones_v)
    plsc.subcore_barrier()

    def fire(j, carry):
        pltpu.async_copy(ones_v, acc_sh.at[dst_v.at[j]], ssem, add=True)
        return carry
    lax.fori_loop(0, NCHUNK, fire, 0)

    def drain(j, carry):
        pltpu.make_async_copy(ones_hbm, ones_v, ssem).wait()
        return carry
    lax.fori_loop(0, NCHUNK, drain, 0)

    plsc.subcore_barrier()
    pltpu.sync_copy(acc_sh.at[pl.ds(s * RPT, RPT)],
                    out_hbm.at[c].at[pl.ds(s * RPT, RPT)])


def _agg_body(rows_hbm, e_hbm, zeros_hbm, out_hbm,
              src_v, dst_v, bufs, acc_sh, gsems, ssems):
    c = lax.axis_index("c")
    s = lax.axis_index("s")
    wid = s * NC + c
    pltpu.sync_copy(zeros_hbm, acc_sh.at[pl.ds(s * RPT, RPT)])
    pltpu.sync_copy(e_hbm.at[0].at[wid], src_v)
    pltpu.sync_copy(e_hbm.at[1].at[wid], dst_v)
    plsc.subcore_barrier()

    def fire_gathers(sb, bi):
        base = sb * SBC
        def f(j, carry):
            pltpu.async_copy(rows_hbm.at[src_v.at[base + j]],
                             bufs.at[bi].at[j], gsems.at[bi])
            return carry
        lax.fori_loop(0, SBC, f, 0)

    def fire_scatters(sb, bi):
        base = sb * SBC
        def f(j, carry):
            pltpu.async_copy(bufs.at[bi].at[j], acc_sh.at[dst_v.at[base + j]],
                             ssems.at[bi], add=True)
            return carry
        lax.fori_loop(0, SBC, f, 0)

    def drain(sem):
        def f(j, carry):
            pltpu.make_async_copy(rows_hbm.at[pl.ds(0, K)],
                                  bufs.at[0].at[0], sem).wait()
            return carry
        lax.fori_loop(0, SBC, f, 0)

    fire_gathers(0, 0)
    for sb in range(NSB):
        bi = sb % 2
        drain(gsems.at[bi])
        if sb >= 1:
            drain(ssems.at[(sb - 1) % 2])
        if sb + 1 < NSB:
            fire_gathers(sb + 1, (sb + 1) % 2)
        fire_scatters(sb, bi)
    drain(ssems.at[(NSB - 1) % 2])

    plsc.subcore_barrier()
    pltpu.sync_copy(acc_sh.at[pl.ds(s * RPT, RPT)],
                    out_hbm.at[c].at[pl.ds(s * RPT, RPT)])


_sc_params = pltpu.CompilerParams(use_tc_tiling_on_sc=False)

_deg_call = pl.kernel(
    _deg_body,
    out_type=jax.ShapeDtypeStruct((NC, NP, HIDDEN), jnp.float32),
    mesh=_sc_mesh(),
    compiler_params=_sc_params,
    scratch_types=[
        pltpu.VMEM((NCHUNK, K), jnp.int32),
        pltpu.VMEM((K, HIDDEN), jnp.float32),
        pltpu.VMEM_SHARED((NP, HIDDEN), jnp.float32),
        pltpu.SemaphoreType.DMA,
    ],
)

_agg_call = pl.kernel(
    _agg_body,
    out_type=jax.ShapeDtypeStruct((NC, NP, HIDDEN), jnp.float32),
    mesh=_sc_mesh(),
    compiler_params=_sc_params,
    scratch_types=[
        pltpu.VMEM((NCHUNK, K), jnp.int32),
        pltpu.VMEM((NCHUNK, K), jnp.int32),
        pltpu.VMEM((2, SBC, K, HIDDEN), jnp.float32),
        pltpu.VMEM_SHARED((NP, HIDDEN), jnp.float32),
        pltpu.SemaphoreType.DMA((2,)),
        pltpu.SemaphoreType.DMA((2,)),
    ],
)


def _tc1_body(xq, w1b, degq, hsq, dinvq):
    dinv = lax.rsqrt(degq[0] + degq[1] + 1.0)
    dinvq[...] = dinv
    h = jnp.dot(xq[...], w1b[...], preferred_element_type=jnp.float32)
    hsq[pl.ds(0, XQ), :] = h * dinv[0:XQ, :]


def _tc2_body(a1p, hsq, dinvq, b1t, gsq):
    a1 = (a1p[0] + a1p[1] + hsq[...]) * dinvq[...] + b1t[...]
    gsq[...] = jnp.maximum(a1, 0.0) * dinvq[...]


def _tc3_body(a2p, gsq, dinvq, w2b, b2t, out):
    a2 = (a2p[0] + a2p[1] + gsq[...]) * dinvq[...]
    o = jnp.dot(a2, w2b[...], preferred_element_type=jnp.float32) + b2t[...]
    out[...] = o[0:XQ, :]


_tc1 = pl.pallas_call(
    _tc1_body,
    out_shape=[jax.ShapeDtypeStruct((NQ, 128), jnp.float32),
               jax.ShapeDtypeStruct((NQ, 128), jnp.float32)],
)

_tc2 = pl.pallas_call(
    _tc2_body,
    out_shape=jax.ShapeDtypeStruct((NQ, 128), jnp.float32),
)

_tc3 = pl.pallas_call(
    _tc3_body,
    out_shape=jax.ShapeDtypeStruct((XQ, 8 * OUT_FEATS), jnp.float32),
)


def kernel(x, edge_index, W1, b1, W2, b2):
    padv = N_NODES + (jnp.arange(EP - N_EDGES, dtype=jnp.int32)
                      % (NP - N_NODES))
    e = jnp.concatenate(
        [edge_index.astype(jnp.int32),
         jnp.broadcast_to(padv, (2, EP - N_EDGES))], axis=1)
    e = e.reshape(2, NW, NCHUNK, K)
    xq = x.reshape(XQ, 8 * IN_FEATS)
    w1b = jnp.kron(jnp.eye(8, dtype=jnp.float32), W1)
    w2b = jnp.kron(jnp.eye(8, dtype=jnp.float32), W2)
    b1t = jnp.tile(b1, 8).reshape(1, 128)
    b2t = jnp.tile(b2, 8).reshape(1, 8 * OUT_FEATS)
    onesKH = jnp.ones((K, HIDDEN), jnp.float32)
    zerosRH = jnp.zeros((RPT, HIDDEN), jnp.float32)

    degp = _deg_call(e, onesKH, zerosRH)
    hsq, dinvq = _tc1(xq, w1b, degp.reshape(NC, NQ, 128))
    a1p = _agg_call(hsq.reshape(NP, HIDDEN), e, zerosRH)
    gsq = _tc2(a1p.reshape(NC, NQ, 128), hsq, dinvq, b1t)
    a2p = _agg_call(gsq.reshape(NP, HIDDEN), e, zerosRH)
    outq = _tc3(a2p.reshape(NC, NQ, 128), gsq, dinvq, w2b, b2t)
    return outq.reshape(N_NODES, OUT_FEATS)


# speedup vs baseline: 1.8300x; 1.0081x over previous
import jax
import jax.numpy as jnp
from jax import lax
from jax.experimental import pallas as pl
from jax.experimental.pallas import tpu as pltpu
from jax.experimental.pallas import tpu_sc as plsc

N_NODES = 10000
N_EDGES = 320000
IN_FEATS = 128
HIDDEN = 16
OUT_FEATS = 2

NC, NS = 2, 16
NW = NC * NS
NP = 10240
NQ = NP // 8
RPT = NP // NS
EPW = N_EDGES // NW
K = 80
NCHUNK = EPW // K
SBC = 25
NSB = NCHUNK // SBC


def _sc_mesh():
    return plsc.VectorSubcoreMesh(
        core_axis_name="c", subcore_axis_name="s",
        num_cores=NC, num_subcores=NS)


def _load_dst_rows(e_hbm, dst_v, wid, lsem):
    base = wid * EPW

    def f(j, carry):
        pltpu.async_copy(e_hbm.at[1].at[pl.ds(base + j * K, K)],
                         dst_v.at[j], lsem)
        return carry
    lax.fori_loop(0, NCHUNK, f, 0)

    def d(j, carry):
        pltpu.make_async_copy(e_hbm.at[1].at[pl.ds(0, K)],
                              dst_v.at[0], lsem).wait()
        return carry
    lax.fori_loop(0, NCHUNK, d, 0)


def _deg_body(e_hbm, ones_hbm, zeros_hbm, out_hbm, dst_v, ones_v, acc_sh,
              ssem):
    c = lax.axis_index("c")
    s = lax.axis_index("s")
    wid = s * NC + c
    pltpu.sync_copy(zeros_hbm, acc_sh.at[pl.ds(s * RPT, RPT)])
    _load_dst_rows(e_hbm, dst_v, wid, ssem)
    pltpu.sync_copy(ones_hbm, ones_v)
    plsc.subcore_barrier()

    def fire(j, carry):
        pltpu.async_copy(ones_v, acc_sh.at[dst_v.at[j]], ssem, add=True)
        return carry
    lax.fori_loop(0, NCHUNK, fire, 0)

    def drain(j, carry):
        pltpu.make_async_copy(ones_hbm, ones_v, ssem).wait()
        return carry
    lax.fori_loop(0, NCHUNK, drain, 0)

    plsc.subcore_barrier()
    pltpu.sync_copy(acc_sh.at[pl.ds(s * RPT, RPT)],
                    out_hbm.at[c].at[pl.ds(s * RPT, RPT)])


def _agg_body(rows_hbm, e_hbm, zeros_hbm, out_hbm,
              src_v, dst_v, bufs, acc_sh, gsems, ssems):
    c = lax.axis_index("c")
    s = lax.axis_index("s")
    wid = s * NC + c
    pltpu.sync_copy(zeros_hbm, acc_sh.at[pl.ds(s * RPT, RPT)])
    pltpu.sync_copy(e_hbm.at[0].at[pl.ds(wid * EPW, EPW)], src_v)
    _load_dst_rows(e_hbm, dst_v, wid, gsems.at[0])
    plsc.subcore_barrier()

    def fire_gathers(sb, bi):
        base = sb * SBC
        def f(j, carry):
            idx = src_v.at[pl.ds((base + j) * K, K)]
            pltpu.async_copy(rows_hbm.at[idx], bufs.at[bi].at[j], gsems.at[bi])
            return carry
        lax.fori_loop(0, SBC, f, 0)

    def fire_scatters(sb, bi):
        base = sb * SBC
        def f(j, carry):
            pltpu.async_copy(bufs.at[bi].at[j], acc_sh.at[dst_v.at[base + j]],
                             ssems.at[bi], add=True)
            return carry
        lax.fori_loop(0, SBC, f, 0)

    def drain(sem):
        def f(j, carry):
            pltpu.make_async_copy(rows_hbm.at[pl.ds(0, K)],
                                  bufs.at[0].at[0], sem).wait()
            return carry
        lax.fori_loop(0, SBC, f, 0)

    fire_gathers(0, 0)
    for sb in range(NSB):
        bi = sb % 2
        drain(gsems.at[bi])
        if sb >= 1:
            drain(ssems.at[(sb - 1) % 2])
        if sb + 1 < NSB:
            fire_gathers(sb + 1, (sb + 1) % 2)
        fire_scatters(sb, bi)
    drain(ssems.at[(NSB - 1) % 2])

    plsc.subcore_barrier()
    pltpu.sync_copy(acc_sh.at[pl.ds(s * RPT, RPT)],
                    out_hbm.at[c].at[pl.ds(s * RPT, RPT)])


_sc_params = pltpu.CompilerParams(use_tc_tiling_on_sc=False)

_deg_call = pl.kernel(
    _deg_body,
    out_type=jax.ShapeDtypeStruct((NC, NP, HIDDEN), jnp.float32),
    mesh=_sc_mesh(),
    compiler_params=_sc_params,
    scratch_types=[
        pltpu.VMEM((NCHUNK, K), jnp.int32),
        pltpu.VMEM((K, HIDDEN), jnp.float32),
        pltpu.VMEM_SHARED((NP, HIDDEN), jnp.float32),
        pltpu.SemaphoreType.DMA,
    ],
)

_agg_call = pl.kernel(
    _agg_body,
    out_type=jax.ShapeDtypeStruct((NC, NP, HIDDEN), jnp.float32),
    mesh=_sc_mesh(),
    compiler_params=_sc_params,
    scratch_types=[
        pltpu.VMEM((EPW,), jnp.int32),
        pltpu.VMEM((NCHUNK, K), jnp.int32),
        pltpu.VMEM((2, SBC, K, HIDDEN), jnp.float32),
        pltpu.VMEM_SHARED((NP, HIDDEN), jnp.float32),
        pltpu.SemaphoreType.DMA((2,)),
        pltpu.SemaphoreType.DMA((2,)),
    ],
)


def _tc1_body(x, w1, degq, hsq, dinvq):
    dinv = lax.rsqrt(degq[0] + degq[1] + 1.0)
    dinvq[...] = dinv
    h = jnp.dot(x[...], w1[...], preferred_element_type=jnp.float32)
    blocks = [h[i * NQ:(i + 1) * NQ, :] for i in range(7)]
    tail = jnp.concatenate(
        [h[7 * NQ:N_NODES, :],
         jnp.zeros((NP - N_NODES, HIDDEN), jnp.float32)], axis=0)
    hp = jnp.concatenate(blocks + [tail], axis=1)
    hsq[...] = hp * dinv


def _tc2_body(a1p, hsq, dinvq, b1t, gsq):
    a1 = (a1p[0] + a1p[1] + hsq[...]) * dinvq[...] + b1t[...]
    gsq[...] = jnp.maximum(a1, 0.0) * dinvq[...]


def _tc3_body(a2p, gsq, dinvq, w2b, b2t, out):
    a2 = (a2p[0] + a2p[1] + gsq[...]) * dinvq[...]
    out[...] = jnp.dot(a2, w2b[...], preferred_element_type=jnp.float32) \
        + b2t[...]


_tc1 = pl.pallas_call(
    _tc1_body,
    out_shape=[jax.ShapeDtypeStruct((NQ, 128), jnp.float32),
               jax.ShapeDtypeStruct((NQ, 128), jnp.float32)],
)

_tc2 = pl.pallas_call(
    _tc2_body,
    out_shape=jax.ShapeDtypeStruct((NQ, 128), jnp.float32),
)

_tc3 = pl.pallas_call(
    _tc3_body,
    out_shape=jax.ShapeDtypeStruct((NQ, 8 * OUT_FEATS), jnp.float32),
)


def kernel(x, edge_index, W1, b1, W2, b2):
    e32 = edge_index.astype(jnp.int32)
    e = (e32 % NQ) * 8 + e32 // NQ
    w2b = jnp.kron(jnp.eye(8, dtype=jnp.float32), W2)
    b1t = jnp.tile(b1, 8).reshape(1, 128)
    b2t = jnp.tile(b2, 8).reshape(1, 8 * OUT_FEATS)
    onesKH = jnp.ones((K, HIDDEN), jnp.float32)
    zerosRH = jnp.zeros((RPT, HIDDEN), jnp.float32)

    degp = _deg_call(e, onesKH, zerosRH)
    hsq, dinvq = _tc1(x, W1, degp.reshape(NC, NQ, 128))
    a1p = _agg_call(hsq.reshape(NP, HIDDEN), e, zerosRH)
    gsq = _tc2(a1p.reshape(NC, NQ, 128), hsq, dinvq, b1t)
    a2p = _agg_call(gsq.reshape(NP, HIDDEN), e, zerosRH)
    outq = _tc3(a2p.reshape(NC, NQ, 128), gsq, dinvq, w2b, b2t)
    o = outq.reshape(NQ, 8, OUT_FEATS).transpose(1, 0, 2)
    return o.reshape(NP, OUT_FEATS)[:N_NODES]
